# Initial kernel scaffold; baseline (speedup 1.0000x reference)
import jax
import jax.numpy as jnp
from jax.experimental import pallas as pl


def kernel(met_x, met_edge_index, met_edge_attr, met_batch, met_fp, sub_x, sub_edge_index, sub_edge_attr, sub_batch, sub_fp, params):
    raise NotImplementedError("write your pallas kernel here")


# speedup vs baseline: 1.0085x; 1.0085x over previous
import jax
import jax.numpy as jnp
from jax.experimental import pallas as pl
from jax.experimental.pallas import tpu as pltpu

N_GRAPHS = 64


def _gatv2(x, src, dst, eattr, p, n):
    esum = jax.ops.segment_sum(eattr, dst, num_segments=n)
    ecnt = jax.ops.segment_sum(jnp.ones((eattr.shape[0], 1), jnp.float32), dst, num_segments=n)
    loop_attr = esum / jnp.maximum(ecnt, 1.0)
    loop = jnp.arange(n, dtype=src.dtype)
    s2 = jnp.concatenate([src, loop])
    d2 = jnp.concatenate([dst, loop])
    ea = jnp.concatenate([eattr, loop_attr], axis=0)
    xl = x @ p["Wl"] + p["bl"]
    xr = x @ p["Wr"] + p["br"]
    xj = xl[s2]
    xi = xr[d2]
    e = jax.nn.leaky_relu(xi + xj + ea @ p["We"], negative_slope=0.2)
    logit = jnp.sum(e * p["att"][None, :], axis=-1)
    mx = jax.ops.segment_max(logit, d2, num_segments=n)
    alpha = jnp.exp(logit - mx[d2])
    denom = jax.ops.segment_sum(alpha, d2, num_segments=n)
    alpha = alpha / jnp.maximum(denom[d2], 1e-16)
    out = jax.ops.segment_sum(alpha[:, None] * xj, d2, num_segments=n)
    return out + p["bias"]


def _branch(x, ei, ea, batch, convs):
    n = x.shape[0]
    h = x
    for p in convs:
        h = jax.nn.relu(_gatv2(h, ei[0], ei[1], ea, p, n))
    s = jax.ops.segment_sum(h, batch, num_segments=N_GRAPHS)
    c = jax.ops.segment_sum(jnp.ones((n,), jnp.float32), batch, num_segments=N_GRAPHS)
    return s / jnp.maximum(c, 1.0)[:, None]


def _fcnn_body(h_ref, w1, b1, g1, bb1, w2, b2, g2, bb2, w3, b3, g3, bb3, w4, b4, o_ref):
    inv = 1.0 / jnp.sqrt(1.0 + 1e-5)
    h = h_ref[...]
    h = jnp.maximum(h @ w1[...] + b1[...], 0.0) * inv * g1[...] + bb1[...]
    h = jnp.maximum(h @ w2[...] + b2[...], 0.0) * inv * g2[...] + bb2[...]
    h = jnp.maximum(h @ w3[...] + b3[...], 0.0) * inv * g3[...] + bb3[...]
    o_ref[...] = jax.nn.sigmoid(h @ w4[...] + b4[...])


def _fcnn(h, params):
    args = []
    for i in (1, 2, 3):
        fc, bn = params[f"fc{i}"], params[f"bn{i}"]
        args += [fc["W"], fc["b"][None, :], bn["g"][None, :], bn["b"][None, :]]
    args += [params["fc4"]["W"], params["fc4"]["b"][None, :]]
    return pl.pallas_call(
        _fcnn_body,
        out_shape=jax.ShapeDtypeStruct((h.shape[0], 1), jnp.float32),
    )(h, *args)


def kernel(met_x, met_edge_index, met_edge_attr, met_batch, met_fp,
           sub_x, sub_edge_index, sub_edge_attr, sub_batch, sub_fp, params):
    node = _branch(met_x, met_edge_index, met_edge_attr, met_batch, params["met"])
    node_sub = _branch(sub_x, sub_edge_index, sub_edge_attr, sub_batch, params["sub"])
    h = jnp.concatenate([node_sub, sub_fp, node, met_fp], axis=1)
    return _fcnn(h, params)


# speedup vs baseline: 1.5534x; 1.5403x over previous
import functools

import jax
import jax.numpy as jnp
from jax import lax
from jax.experimental import pallas as pl
from jax.experimental.pallas import tpu as pltpu
from jax.experimental.pallas import tpu_sc as plsc

N_GRAPHS = 64
N = 10000
E = 160000
E2 = E + N
NSUB = 16
NP = 10240
NPT = NP // NSUB
EP = 163840
EP2 = 172032
NBLK2 = EP2 // (NSUB * 128)
DIMS = [(10, 300), (300, 60), (60, 730), (730, 370)]
NEG = -3e38

_SC_MESH = plsc.VectorSubcoreMesh(core_axis_name="c", subcore_axis_name="s")
_SC_PARAMS = pltpu.CompilerParams(needs_layout_passes=False)


def _pad128(d):
    return ((d + 127) // 128) * 128


def _iota16():
    return lax.broadcasted_iota(jnp.int32, (16,), 0)


def _loopsum_body(ea16, dst3, out, idxbuf, rowbuf, accum):
    c = lax.axis_index("c")
    s = lax.axis_index("s")
    nblk = ea16.shape[1] // (NSUB * 128)

    def zrow(i, _):
        for j in range(8):
            rowbuf[i, pl.ds(j * 16, 16)] = jnp.zeros((16,), jnp.float32)
        return 0
    lax.fori_loop(0, 128, zrow, 0)
    for k in range(NPT // 128):
        pltpu.sync_copy(rowbuf, accum.at[pl.ds(s * NPT + k * 128, 128)])
    plsc.subcore_barrier()

    def blk(i, _):
        row = s * nblk + i
        pltpu.sync_copy(dst3.at[c, row], idxbuf)
        pltpu.sync_copy(ea16.at[c, pl.ds(row * 128, 128)], rowbuf)
        pltpu.sync_copy(rowbuf, accum.at[idxbuf], add=True)
        return 0

    lax.fori_loop(0, nblk, blk, 0)
    plsc.subcore_barrier()
    for k in range(NPT // 128):
        pltpu.sync_copy(accum.at[pl.ds(s * NPT + k * 128, 128)], rowbuf)
        pltpu.sync_copy(rowbuf, out.at[c, pl.ds(s * NPT + k * 128, 128)])


def _loopsum(ea16, dst3):
    return pl.kernel(
        _loopsum_body,
        out_type=jax.ShapeDtypeStruct((2, NP, 128), jnp.float32),
        mesh=_SC_MESH,
        compiler_params=_SC_PARAMS,
        scratch_types=[
            pltpu.VMEM((128,), jnp.int32),
            pltpu.VMEM((128, 128), jnp.float32),
            pltpu.VMEM_SHARED((NP, 128), jnp.float32),
        ],
    )(ea16, dst3)


def _logit_body(ncw, xlr3, ew3, s2r, d2r, attp, logits,
                sidx, didx, gl, gr, gew, abuf, acc, lblk):
    c = lax.axis_index("c")
    s = lax.axis_index("s")
    pltpu.sync_copy(attp.at[c], abuf)

    def blk(i, _):
        row = s * NBLK2 + i
        pltpu.sync_copy(s2r.at[c, row], sidx)
        pltpu.sync_copy(d2r.at[c, row], didx)

        def zero_acc(e, _):
            acc[e, :] = jnp.zeros((16,), jnp.float32)
            return 0
        lax.fori_loop(0, 128, zero_acc, 0)

        for cc in range(ncw):
            pltpu.sync_copy(xlr3.at[c * 2 * ncw + cc].at[sidx], gl)
            pltpu.sync_copy(xlr3.at[c * 2 * ncw + ncw + cc].at[didx], gr)
            pltpu.sync_copy(ew3.at[c * ncw + cc, pl.ds(row * 128, 128)], gew)

            def edge(e, _):
                a = acc[e, :]
                for j in range(8):
                    u = (gl[e, pl.ds(j * 16, 16)] + gr[e, pl.ds(j * 16, 16)]
                         + gew[e, pl.ds(j * 16, 16)])
                    u = jnp.where(u > 0.0, u, 0.2 * u)
                    a = a + abuf[pl.ds(cc * 128 + j * 16, 16)] * u
                acc[e, :] = a
                return 0
            lax.fori_loop(0, 128, edge, 0)

        base = row * 128
        i16 = _iota16()

        def fin(g, _):
            erows = g * 16 + i16
            tot = plsc.load_gather(acc, [erows, jnp.zeros((16,), jnp.int32)])
            for l in range(1, 16):
                tot = tot + plsc.load_gather(
                    acc, [erows, jnp.full((16,), l, jnp.int32)])
            eidv = base + erows
            tot = jnp.where(eidv < E2, tot, NEG)
            lblk[pl.ds(g * 16, 16)] = tot
            return 0
        lax.fori_loop(0, 8, fin, 0)
        pltpu.sync_copy(lblk, logits.at[c, pl.ds(base, 128)])
        return 0

    lax.fori_loop(0, NBLK2, blk, 0)


def _logit(xlr3, ew3, s2r, d2r, attp):
    ncw = ew3.shape[0] // 2
    xlr3 = xlr3.reshape(-1, NP, 128)
    return pl.kernel(
        functools.partial(_logit_body, ncw),
        out_type=jax.ShapeDtypeStruct((2, EP2), jnp.float32),
        mesh=_SC_MESH,
        compiler_params=_SC_PARAMS,
        scratch_types=[
            pltpu.VMEM((128,), jnp.int32),
            pltpu.VMEM((128,), jnp.int32),
            pltpu.VMEM((128, 128), jnp.float32),
            pltpu.VMEM((128, 128), jnp.float32),
            pltpu.VMEM((128, 128), jnp.float32),
            pltpu.VMEM((ncw * 128,), jnp.float32),
            pltpu.VMEM((128, 16), jnp.float32),
            pltpu.VMEM((128,), jnp.float32),
        ],
    )(xlr3, ew3, s2r, d2r, attp)


def _mxden_body(logits, d2r, mx, den, m16, d16, red, lblk, d2b):
    c = lax.axis_index("c")
    s = lax.axis_index("s")
    lo = s * NPT
    i16 = _iota16()

    def init(i, _):
        m16[pl.ds(i * 16, 16)] = jnp.full((16,), NEG, jnp.float32)
        d16[pl.ds(i * 16, 16)] = jnp.zeros((16,), jnp.float32)
        return 0
    lax.fori_loop(0, NPT, init, 0)

    trash = NPT * 16
    def scan_max(r, _):
        pltpu.sync_copy(d2r.at[c, r], d2b)
        pltpu.sync_copy(logits.at[c, pl.ds(r * 128, 128)], lblk)
        for v in range(8):
            d2v = d2b[pl.ds(v * 16, 16)] - lo
            lv = lblk[pl.ds(v * 16, 16)]
            msk = (d2v >= 0) & (d2v < NPT)
            idx = jnp.where(msk, d2v * 16 + i16, trash + i16)
            cur = plsc.load_gather(m16, [idx])
            plsc.store_scatter(m16, [idx], jnp.maximum(cur, lv))
        return 0
    lax.fori_loop(0, EP2 // 128, scan_max, 0)

    def red_max(g, _):
        rows = (g * 16 + i16) * 16
        acc = plsc.load_gather(m16, [rows])
        for l in range(1, 16):
            acc = jnp.maximum(acc, plsc.load_gather(m16, [rows + l]))
        red[pl.ds(g * 16, 16)] = acc
        return 0
    lax.fori_loop(0, NPT // 16, red_max, 0)
    pltpu.sync_copy(red, mx.at[c, pl.ds(lo, NPT)])

    def scan_den(r, _):
        pltpu.sync_copy(d2r.at[c, r], d2b)
        pltpu.sync_copy(logits.at[c, pl.ds(r * 128, 128)], lblk)
        for v in range(8):
            d2v = d2b[pl.ds(v * 16, 16)] - lo
            lv = lblk[pl.ds(v * 16, 16)]
            msk = (d2v >= 0) & (d2v < NPT)
            safe = jnp.where(msk, d2v, 0)
            mv = plsc.load_gather(red, [safe])
            a = jnp.minimum(jnp.exp(lv - mv), 3e38)
            idx = jnp.where(msk, safe * 16 + i16, trash + i16)
            plsc.addupdate_scatter(d16, [idx], a)
        return 0
    lax.fori_loop(0, EP2 // 128, scan_den, 0)

    def red_den(g, _):
        rows = (g * 16 + i16) * 16
        acc = plsc.load_gather(d16, [rows])
        for l in range(1, 16):
            acc = acc + plsc.load_gather(d16, [rows + l])
        red[pl.ds(g * 16, 16)] = acc
        return 0
    lax.fori_loop(0, NPT // 16, red_den, 0)
    pltpu.sync_copy(red, den.at[c, pl.ds(lo, NPT)])


def _mxden(logits, d2r):
    return pl.kernel(
        _mxden_body,
        out_type=(jax.ShapeDtypeStruct((2, NP), jnp.float32),
                  jax.ShapeDtypeStruct((2, NP), jnp.float32)),
        mesh=_SC_MESH,
        compiler_params=_SC_PARAMS,
        scratch_types=[
            pltpu.VMEM((NPT * 16 + 16,), jnp.float32),
            pltpu.VMEM((NPT * 16 + 16,), jnp.float32),
            pltpu.VMEM((NPT,), jnp.float32),
            pltpu.VMEM((128,), jnp.float32),
            pltpu.VMEM((128,), jnp.int32),
        ],
    )(logits, d2r)


def _alpha_body(logits, d2r, mx, den, alphan, mxall, denall, lblk, d2b, ablk):
    c = lax.axis_index("c")
    s = lax.axis_index("s")
    pltpu.sync_copy(mx.at[c], mxall)
    pltpu.sync_copy(den.at[c], denall)

    def blk(i, _):
        row = s * NBLK2 + i
        pltpu.sync_copy(d2r.at[c, row], d2b)
        pltpu.sync_copy(logits.at[c, pl.ds(row * 128, 128)], lblk)
        for v in range(8):
            d2v = d2b[pl.ds(v * 16, 16)]
            lv = lblk[pl.ds(v * 16, 16)]
            mv = plsc.load_gather(mxall, [d2v])
            dv = plsc.load_gather(denall, [d2v])
            a = jnp.exp(lv - mv) / jnp.maximum(dv, 1e-16)
            ablk[pl.ds(v * 16, 16)] = a
        pltpu.sync_copy(ablk, alphan.at[c, pl.ds(row * 128, 128)])
        return 0
    lax.fori_loop(0, NBLK2, blk, 0)


def _alpha(logits, d2r, mx, den):
    return pl.kernel(
        _alpha_body,
        out_type=jax.ShapeDtypeStruct((2, EP2), jnp.float32),
        mesh=_SC_MESH,
        compiler_params=_SC_PARAMS,
        scratch_types=[
            pltpu.VMEM((NP,), jnp.float32),
            pltpu.VMEM((NP,), jnp.float32),
            pltpu.VMEM((128,), jnp.float32),
            pltpu.VMEM((128,), jnp.int32),
            pltpu.VMEM((128,), jnp.float32),
        ],
    )(logits, d2r, mx, den)


def _outacc_body(ncw, xlr3, alphan, s2r, d2r, out3,
                 sidx, didx, gbuf, ablk, accum):
    c = lax.axis_index("c")
    s = lax.axis_index("s")
    z16 = jnp.zeros((16,), jnp.int32)

    for cc in range(ncw):
        def zrow(i, _):
            for j in range(8):
                gbuf[i, pl.ds(j * 16, 16)] = jnp.zeros((16,), jnp.float32)
            return 0
        lax.fori_loop(0, 128, zrow, 0)
        for k in range(NPT // 128):
            pltpu.sync_copy(gbuf, accum.at[pl.ds(s * NPT + k * 128, 128)])
        plsc.subcore_barrier()

        def blk(i, _):
            row = s * NBLK2 + i
            pltpu.sync_copy(s2r.at[c, row], sidx)
            pltpu.sync_copy(d2r.at[c, row], didx)
            pltpu.sync_copy(alphan.at[c, pl.ds(row * 128, 128)], ablk)
            pltpu.sync_copy(xlr3.at[c * 2 * ncw + cc].at[sidx], gbuf)

            def edge(e, _):
                av = plsc.load_gather(ablk, [z16 + e])
                for j in range(8):
                    gbuf[e, pl.ds(j * 16, 16)] = av * gbuf[e, pl.ds(j * 16, 16)]
                return 0
            lax.fori_loop(0, 128, edge, 0)
            pltpu.sync_copy(gbuf, accum.at[didx], add=True)
            return 0
        lax.fori_loop(0, NBLK2, blk, 0)
        plsc.subcore_barrier()
        for k in range(NPT // 128):
            pltpu.sync_copy(accum.at[pl.ds(s * NPT + k * 128, 128)], gbuf)
            pltpu.sync_copy(gbuf, out3.at[c * ncw + cc, pl.ds(s * NPT + k * 128, 128)])
        plsc.subcore_barrier()


def _outacc(xlr3, alphan, s2r, d2r):
    ncw = xlr3.shape[1] // 2
    xlr3 = xlr3.reshape(-1, NP, 128)
    return pl.kernel(
        functools.partial(_outacc_body, ncw),
        out_type=jax.ShapeDtypeStruct((2 * ncw, NP, 128), jnp.float32),
        mesh=_SC_MESH,
        compiler_params=_SC_PARAMS,
        scratch_types=[
            pltpu.VMEM((128,), jnp.int32),
            pltpu.VMEM((128,), jnp.int32),
            pltpu.VMEM((128, 128), jnp.float32),
            pltpu.VMEM((128,), jnp.float32),
            pltpu.VMEM_SHARED((NP, 128), jnp.float32),
        ],
    )(xlr3, alphan, s2r, d2r)


def _proj_body(use_relu, h_ref, w_ref, b_ref, pb_ref, o_ref):
    h = h_ref[0]
    if use_relu:
        h = jnp.maximum(h + pb_ref[0], 0.0)
    o_ref[0, 0] = h @ w_ref[0] + b_ref[0]


def _proj(h, wcat, bcat, prev_bias, use_relu):
    _, _, din_p = h.shape
    two_ncw = wcat.shape[2] // 128
    bn = 1024
    grid = (2, NP // bn, two_ncw)
    return pl.pallas_call(
        functools.partial(_proj_body, use_relu),
        grid=grid,
        in_specs=[
            pl.BlockSpec((1, bn, din_p), lambda b, i, j: (b, i, 0)),
            pl.BlockSpec((1, din_p, 128), lambda b, i, j: (b, 0, j)),
            pl.BlockSpec((1, 1, 128), lambda b, i, j: (b, 0, j)),
            pl.BlockSpec((1, 1, din_p), lambda b, i, j: (b, 0, 0)),
        ],
        out_specs=pl.BlockSpec((1, 1, bn, 128), lambda b, i, j: (b, j, i, 0)),
        out_shape=jax.ShapeDtypeStruct((2, two_ncw, NP, 128), jnp.float32),
    )(h, wcat, bcat, prev_bias)


def _ew_body(ea_ref, w_ref, o_ref):
    o_ref[0] = ea_ref[0] @ w_ref[0]


def _ew(ea2, wep):
    ncw = wep.shape[2] // 128
    be = 2048
    grid = (2, EP2 // be, ncw)
    return pl.pallas_call(
        _ew_body,
        grid=grid,
        in_specs=[
            pl.BlockSpec((1, be, 16), lambda b, i, j: (b, i, 0)),
            pl.BlockSpec((1, 16, 128), lambda b, i, j: (b, 0, j)),
        ],
        out_specs=pl.BlockSpec((1, be, 128), lambda b, i, j: (b * ncw + j, i, 0)),
        out_shape=jax.ShapeDtypeStruct((2 * ncw, EP2, 128), jnp.float32),
    )(ea2, wep)


def _pool_body(h_ref, b_ref, bias_ref, ps_ref, pc_ref):
    i = pl.program_id(1)

    @pl.when(i == 0)
    def _():
        ps_ref[...] = jnp.zeros_like(ps_ref)
        pc_ref[...] = jnp.zeros_like(pc_ref)

    h = jnp.maximum(h_ref[0] + bias_ref[0], 0.0)
    onehot = (b_ref[0] == lax.broadcasted_iota(jnp.int32, (1, N_GRAPHS), 1)
              ).astype(jnp.float32)
    ps_ref[0] += onehot.T @ h
    pc_ref[0] += onehot.T @ jnp.ones((h.shape[0], 8), jnp.float32)


def _pool(h4, batch2, bias4):
    bn = 1024
    dp = h4.shape[2]
    grid = (2, NP // bn)
    return pl.pallas_call(
        _pool_body,
        grid=grid,
        in_specs=[
            pl.BlockSpec((1, bn, dp), lambda b, i: (b, i, 0)),
            pl.BlockSpec((1, bn, 1), lambda b, i: (b, i, 0)),
            pl.BlockSpec((1, 1, dp), lambda b, i: (b, 0, 0)),
        ],
        out_specs=(
            pl.BlockSpec((1, N_GRAPHS, dp), lambda b, i: (b, 0, 0)),
            pl.BlockSpec((1, N_GRAPHS, 8), lambda b, i: (b, 0, 0)),
        ),
        out_shape=(
            jax.ShapeDtypeStruct((2, N_GRAPHS, dp), jnp.float32),
            jax.ShapeDtypeStruct((2, N_GRAPHS, 8), jnp.float32),
        ),
    )(h4, batch2, bias4)


def _fcnn_body(ps_ref, pc_ref, mfp, sfp,
               w1, b1, g1, bb1, w2, b2, g2, bb2, w3, b3, g3, bb3, w4, b4,
               o_ref):
    inv = 1.0 / jnp.sqrt(1.0 + 1e-5)
    node = ps_ref[0, :, :370] / jnp.maximum(pc_ref[0, :, :1], 1.0)
    node_sub = ps_ref[1, :, :370] / jnp.maximum(pc_ref[1, :, :1], 1.0)
    h = jnp.concatenate([node_sub, sfp[...], node, mfp[...]], axis=1)
    h = jnp.maximum(h @ w1[...] + b1[...], 0.0) * inv * g1[...] + bb1[...]
    h = jnp.maximum(h @ w2[...] + b2[...], 0.0) * inv * g2[...] + bb2[...]
    h = jnp.maximum(h @ w3[...] + b3[...], 0.0) * inv * g3[...] + bb3[...]
    o_ref[...] = jax.nn.sigmoid(h @ w4[...] + b4[...])


def _fcnn(psum, pcnt, met_fp, sub_fp, params):
    args = [psum, pcnt, met_fp, sub_fp]
    for i in (1, 2, 3):
        fc, bn = params[f"fc{i}"], params[f"bn{i}"]
        args += [fc["W"], fc["b"][None, :], bn["g"][None, :], bn["b"][None, :]]
    args += [params["fc4"]["W"], params["fc4"]["b"][None, :]]
    return pl.pallas_call(
        _fcnn_body,
        out_shape=jax.ShapeDtypeStruct((N_GRAPHS, 1), jnp.float32),
    )(*args)


def _prep_weights(params):
    per_layer = []
    for li, (din, dout) in enumerate(DIMS):
        din_p, dout_p = _pad128(din), _pad128(dout)
        wcat = jnp.zeros((2, din_p, 2 * dout_p), jnp.float32)
        bcat = jnp.zeros((2, 1, 2 * dout_p), jnp.float32)
        wep = jnp.zeros((2, 16, dout_p), jnp.float32)
        attp = jnp.zeros((2, dout_p), jnp.float32)
        bias = jnp.zeros((2, 1, dout_p), jnp.float32)
        for b, br in enumerate(("met", "sub")):
            p = params[br][li]
            wcat = wcat.at[b, :din, :dout].set(p["Wl"])
            wcat = wcat.at[b, :din, dout_p:dout_p + dout].set(p["Wr"])
            bcat = bcat.at[b, 0, :dout].set(p["bl"])
            bcat = bcat.at[b, 0, dout_p:dout_p + dout].set(p["br"])
            wep = wep.at[b, :6, :dout].set(p["We"])
            attp = attp.at[b].set(jnp.pad(p["att"], (0, dout_p - dout)))
            bias = bias.at[b, 0, :dout].set(p["bias"])
        per_layer.append((wcat, bcat, wep, attp, bias))
    return per_layer


def kernel(met_x, met_edge_index, met_edge_attr, met_batch, met_fp,
           sub_x, sub_edge_index, sub_edge_attr, sub_batch, sub_fp, params):
    f32 = jnp.float32

    def prep_ls(eattr):
        ea = jnp.zeros((EP, 128), f32)
        ea = ea.at[:E, :6].set(eattr)
        ea = ea.at[:E, 6].set(1.0)
        return ea
    ea16 = jnp.stack([prep_ls(met_edge_attr), prep_ls(sub_edge_attr)])
    dstp = jnp.stack([
        jnp.pad(met_edge_index[1], (0, EP - E)),
        jnp.pad(sub_edge_index[1], (0, EP - E)),
    ]).reshape(2, EP // 128, 128)
    loopsum = _loopsum(ea16, dstp)
    loop_attr = loopsum[:, :N, :6] / jnp.maximum(loopsum[:, :N, 6:7], 1.0)

    loop = jnp.arange(N, dtype=jnp.int32)
    pad2 = EP2 - E2

    def prep_idx(ei_row):
        return jnp.pad(jnp.concatenate([ei_row, loop]), (0, pad2))
    s2r = jnp.stack([prep_idx(met_edge_index[0]),
                     prep_idx(sub_edge_index[0])]).reshape(2, EP2 // 128, 128)
    d2r = jnp.stack([prep_idx(met_edge_index[1]),
                     prep_idx(sub_edge_index[1])]).reshape(2, EP2 // 128, 128)

    def prep_ea2(eattr, la):
        ea = jnp.zeros((EP2, 16), f32)
        ea = ea.at[:E, :6].set(eattr)
        ea = ea.at[E:E2, :6].set(la)
        return ea
    ea2 = jnp.stack([prep_ea2(met_edge_attr, loop_attr[0]),
                     prep_ea2(sub_edge_attr, loop_attr[1])])

    wl = _prep_weights(params)

    h = jnp.zeros((2, NP, 128), f32)
    h = h.at[0, :N, :10].set(met_x)
    h = h.at[1, :N, :10].set(sub_x)
    prev_bias = jnp.zeros((2, 1, 128), f32)

    for li in range(4):
        wcat, bcat, wep, attp, bias = wl[li]
        xlr3 = _proj(h, wcat, bcat, prev_bias, use_relu=(li > 0))
        ew3 = _ew(ea2, wep)
        logits = _logit(xlr3, ew3, s2r, d2r, attp)
        mx, den = _mxden(logits, d2r)
        alphan = _alpha(logits, d2r, mx, den)
        out3 = _outacc(xlr3, alphan, s2r, d2r)
        ncw = out3.shape[0] // 2
        out4 = out3.reshape(2, ncw, NP, 128)
        h = jnp.transpose(out4, (0, 2, 1, 3)).reshape(2, NP, ncw * 128)
        prev_bias = bias

    batch2 = jnp.stack([
        jnp.pad(met_batch, (0, NP - N), constant_values=N_GRAPHS),
        jnp.pad(sub_batch, (0, NP - N), constant_values=N_GRAPHS),
    ]).reshape(2, NP, 1)
    psum, pcnt = _pool(h, batch2, prev_bias[:, :, :h.shape[2]])
    return _fcnn(psum, pcnt, met_fp, sub_fp, params)
